# baseline (speedup 1.0000x reference)
import functools

import jax
import jax.numpy as jnp
from jax import lax
from jax.experimental import pallas as pl
from jax.experimental.pallas import tpu as pltpu

SEQ_LEN = 8192
EMB_DIM = 1024
BATCH = 4
B_TOTAL = BATCH * SEQ_LEN
ROWS_PER_BLOCK = 256
GRID = B_TOTAL // ROWS_PER_BLOCK


def _tc_body(seq_smem, table_ref, out_ref):
    i = pl.program_id(0)

    def row_step(k, carry):
        idx = seq_smem[i * ROWS_PER_BLOCK + k]
        out_ref[k] = table_ref[idx]
        return carry

    lax.fori_loop(0, ROWS_PER_BLOCK, row_step, 0)


@jax.jit
def _positional_encoding(seq_flat, table3):
    grid_spec = pltpu.PrefetchScalarGridSpec(
        num_scalar_prefetch=1,
        grid=(GRID,),
        in_specs=[
            pl.BlockSpec((SEQ_LEN, 8, 128), lambda i, seq: (0, 0, 0)),
        ],
        out_specs=pl.BlockSpec(
            (ROWS_PER_BLOCK, 8, 128), lambda i, seq: (i, 0, 0)),
    )
    return pl.pallas_call(
        _tc_body,
        grid_spec=grid_spec,
        out_shape=jax.ShapeDtypeStruct((B_TOTAL, 8, 128), jnp.float32),
    )(seq_flat, table3)


def kernel(seq, position_embed):
    seq_flat = seq.reshape(B_TOTAL).astype(jnp.int32)
    table3 = position_embed.reshape(SEQ_LEN, 8, 128)
    out = _positional_encoding(seq_flat, table3)
    return out.reshape(BATCH, SEQ_LEN, EMB_DIM)


# speedup vs baseline: 1.5070x; 1.5070x over previous
import functools

import jax
import jax.numpy as jnp
from jax import lax
from jax.experimental import pallas as pl
from jax.experimental.pallas import tpu as pltpu

SEQ_LEN = 8192
EMB_DIM = 1024
BATCH = 4
B_TOTAL = BATCH * SEQ_LEN
ROWS_PER_BLOCK = 256
GRID = B_TOTAL // ROWS_PER_BLOCK


UNROLL = 16


def _tc_body(seq_smem, table_ref, out_ref):
    i = pl.program_id(0)

    def row_step(k, carry):
        base = i * ROWS_PER_BLOCK + k * UNROLL
        for u in range(UNROLL):
            idx = seq_smem[base + u]
            out_ref[k * UNROLL + u] = table_ref[idx]
        return carry

    lax.fori_loop(0, ROWS_PER_BLOCK // UNROLL, row_step, 0)


@jax.jit
def _positional_encoding(seq_flat, table3):
    grid_spec = pltpu.PrefetchScalarGridSpec(
        num_scalar_prefetch=1,
        grid=(GRID,),
        in_specs=[
            pl.BlockSpec((SEQ_LEN, 8, 128), lambda i, seq: (0, 0, 0)),
        ],
        out_specs=pl.BlockSpec(
            (ROWS_PER_BLOCK, 8, 128), lambda i, seq: (i, 0, 0)),
    )
    return pl.pallas_call(
        _tc_body,
        grid_spec=grid_spec,
        out_shape=jax.ShapeDtypeStruct((B_TOTAL, 8, 128), jnp.float32),
    )(seq_flat, table3)


def kernel(seq, position_embed):
    seq_flat = seq.reshape(B_TOTAL).astype(jnp.int32)
    table3 = position_embed.reshape(SEQ_LEN, 8, 128)
    out = _positional_encoding(seq_flat, table3)
    return out.reshape(BATCH, SEQ_LEN, EMB_DIM)


# speedup vs baseline: 3.7589x; 2.4943x over previous
import functools

import jax
import jax.numpy as jnp
from jax import lax
from jax.experimental import pallas as pl
from jax.experimental.pallas import tpu as pltpu
from jax.experimental.pallas import tpu_sc as plsc

SEQ_LEN = 8192
EMB_DIM = 1024
BATCH = 4

NUM_CORES = 2
NUM_SUBCORES = 16
NW = NUM_CORES * NUM_SUBCORES
B_TOTAL = BATCH * SEQ_LEN
B_PER_W = B_TOTAL // NW
CHUNK = 32
N_CHUNKS = B_PER_W // CHUNK
NBUF = 3


def _gather_body(seq_hbm, table_hbm, out_hbm, idx_v, rows_v, sem_g, sem_w):
    wid = lax.axis_index("s") * NUM_CORES + lax.axis_index("c")
    base = wid * B_PER_W

    pltpu.sync_copy(seq_hbm.at[wid], idx_v)

    def fire_gather(j):
        pltpu.async_copy(
            table_hbm.at[idx_v.at[j]], rows_v.at[lax.rem(j, NBUF)], sem_g)

    def wait_gather(j):
        pltpu.make_async_copy(
            table_hbm.at[idx_v.at[j]], rows_v.at[lax.rem(j, NBUF)],
            sem_g).wait()

    def fire_write(j):
        pltpu.async_copy(
            rows_v.at[lax.rem(j, NBUF)],
            out_hbm.at[pl.ds(base + j * CHUNK, CHUNK)], sem_w)

    def wait_write(j):
        pltpu.make_async_copy(
            rows_v.at[lax.rem(j, NBUF)],
            out_hbm.at[pl.ds(base + j * CHUNK, CHUNK)], sem_w).wait()

    fire_gather(0)
    fire_gather(1)

    def step(j, carry):
        wait_gather(j)
        @pl.when(j >= 1)
        def _():
            wait_write(j - 1)

        @pl.when(j + 2 < N_CHUNKS)
        def _():
            fire_gather(j + 2)

        fire_write(j)
        return carry

    lax.fori_loop(0, N_CHUNKS, step, 0)
    wait_write(N_CHUNKS - 1)


@jax.jit
def _positional_encoding(seq_grouped, position_embed):
    mesh = plsc.VectorSubcoreMesh(core_axis_name="c", subcore_axis_name="s")
    run = pl.kernel(
        _gather_body,
        out_type=jax.ShapeDtypeStruct((B_TOTAL, EMB_DIM), jnp.float32),
        mesh=mesh,
        scratch_types=[
            pltpu.VMEM((N_CHUNKS, CHUNK), jnp.int32),
            pltpu.VMEM((NBUF, CHUNK, EMB_DIM), jnp.float32),
            pltpu.SemaphoreType.DMA,
            pltpu.SemaphoreType.DMA,
        ],
    )
    return run(seq_grouped, position_embed)


def kernel(seq, position_embed):
    seq_grouped = seq.reshape(NW, N_CHUNKS, CHUNK).astype(jnp.int32)
    out = _positional_encoding(seq_grouped, position_embed)
    return out.reshape(BATCH, SEQ_LEN, EMB_DIM)
